# Initial kernel scaffold; baseline (speedup 1.0000x reference)
import functools

import jax
import jax.numpy as jnp
from jax import lax
from jax.experimental import pallas as pl
from jax.experimental.pallas import tpu as pltpu
from jax.experimental.pallas import tpu_sc as plsc

LANES = 16


def _make_gather_sum(n_out, max_nb, hidden, chunk):
    mesh = plsc.VectorSubcoreMesh(core_axis_name="c", subcore_axis_name="s")
    nw = mesh.num_cores * mesh.num_subcores
    per_worker = n_out // nw
    assert per_worker * nw == n_out
    n_chunks = per_worker // chunk
    assert n_chunks * chunk == per_worker
    idx_len = chunk * max_nb
    assert idx_len <= 128 and idx_len % 8 == 0

    @functools.partial(
        pl.kernel,
        out_type=jax.ShapeDtypeStruct((n_out, hidden), jnp.float32),
        mesh=mesh,
        scratch_types=[
            pltpu.VMEM((idx_len,), jnp.int32),
            pltpu.VMEM((idx_len, hidden), jnp.float32),
            pltpu.VMEM((chunk, hidden), jnp.float32),
            pltpu.SemaphoreType.DMA,
        ],
    )
    def gather_sum(idx_hbm, table_hbm, out_hbm, idx_v, rows_v, out_v, sem):
        wid = lax.axis_index("s") * mesh.num_cores + lax.axis_index("c")
        w_base = wid * per_worker

        def body(t, carry):
            base = w_base + t * chunk
            pltpu.sync_copy(idx_hbm.at[pl.ds(base * max_nb, idx_len)], idx_v)
            pltpu.async_copy(table_hbm.at[idx_v], rows_v, sem).wait()
            for i in range(chunk):
                for l in range(hidden // LANES):
                    sl = pl.ds(l * LANES, LANES)
                    acc = rows_v[i * max_nb, sl]
                    for j in range(1, max_nb):
                        acc = acc + rows_v[i * max_nb + j, sl]
                    out_v[i, sl] = acc
            pltpu.sync_copy(out_v, out_hbm.at[pl.ds(base, chunk)])
            return carry

        lax.fori_loop(0, n_chunks, body, 0)

    return gather_sum


def _init_body(fb_ref, wt_ref, bin_ref, msg_ref):
    b = jnp.dot(fb_ref[...], wt_ref[...], preferred_element_type=jnp.float32)
    bin_ref[...] = b
    msg_ref[...] = jnp.maximum(b, 0.0)


def _step_body(nei_ref, bin_ref, wt_ref, msg_ref):
    h = jnp.dot(nei_ref[...], wt_ref[...], preferred_element_type=jnp.float32)
    msg_ref[...] = jnp.maximum(bin_ref[...] + h, 0.0)


def _readout_body(fa_ref, nei_ref, wat_ref, wht_ref, b_ref, out_ref, *, apm, mols_blk):
    h = jnp.dot(fa_ref[...], wat_ref[...], preferred_element_type=jnp.float32)
    h += jnp.dot(nei_ref[...], wht_ref[...], preferred_element_type=jnp.float32)
    h = jnp.maximum(h + b_ref[...], 0.0)
    rows = lax.broadcasted_iota(jnp.int32, (mols_blk, mols_blk * apm), 0)
    cols = lax.broadcasted_iota(jnp.int32, (mols_blk, mols_blk * apm), 1) // apm
    seg = (rows == cols).astype(jnp.float32)
    out_ref[...] = jnp.dot(seg, h, preferred_element_type=jnp.float32)


def kernel(fatoms, fbonds, agraph, bgraph, scope_starts, scope_lengths,
           W_i, W_h, W_o_w, W_o_b):
    n_atoms, atom_fdim = fatoms.shape
    n_bonds, bond_in = fbonds.shape
    max_nb = bgraph.shape[1]
    hidden = W_h.shape[0]
    n_mols = scope_starts.shape[0]
    apm = n_atoms // n_mols
    depth = 4

    bflat = bgraph.reshape(-1)
    n_atoms_pad = ((n_atoms + 32 * 20 - 1) // (32 * 20)) * (32 * 20)
    aflat = jnp.concatenate(
        [agraph.reshape(-1),
         jnp.zeros(((n_atoms_pad - n_atoms) * max_nb,), dtype=jnp.int32)])
    W_iT = W_i.T
    W_hT = W_h.T
    W_o_aT = W_o_w[:, :atom_fdim].T
    W_o_hT = W_o_w[:, atom_fdim:].T
    bias = W_o_b.reshape(1, hidden)

    gather_bonds = _make_gather_sum(n_bonds, max_nb, hidden, chunk=20)
    gather_atoms = _make_gather_sum(n_atoms_pad, max_nb, hidden, chunk=20)

    blk = 1000
    grid_b = n_bonds // blk
    binput, message = pl.pallas_call(
        _init_body,
        grid=(grid_b,),
        in_specs=[
            pl.BlockSpec((blk, bond_in), lambda i: (i, 0)),
            pl.BlockSpec((bond_in, hidden), lambda i: (0, 0)),
        ],
        out_specs=[pl.BlockSpec((blk, hidden), lambda i: (i, 0))] * 2,
        out_shape=[jax.ShapeDtypeStruct((n_bonds, hidden), jnp.float32)] * 2,
    )(fbonds, W_iT)

    step_call = pl.pallas_call(
        _step_body,
        grid=(grid_b,),
        in_specs=[
            pl.BlockSpec((blk, hidden), lambda i: (i, 0)),
            pl.BlockSpec((blk, hidden), lambda i: (i, 0)),
            pl.BlockSpec((hidden, hidden), lambda i: (0, 0)),
        ],
        out_specs=pl.BlockSpec((blk, hidden), lambda i: (i, 0)),
        out_shape=jax.ShapeDtypeStruct((n_bonds, hidden), jnp.float32),
    )

    for _ in range(depth - 1):
        nei = gather_bonds(bflat, message)
        message = step_call(nei, binput, W_hT)

    nei_atoms = gather_atoms(aflat, message)[:n_atoms]

    mols_blk = 20
    atoms_blk = mols_blk * apm
    grid_a = n_atoms // atoms_blk
    sums = pl.pallas_call(
        functools.partial(_readout_body, apm=apm, mols_blk=mols_blk),
        grid=(grid_a,),
        in_specs=[
            pl.BlockSpec((atoms_blk, atom_fdim), lambda i: (i, 0)),
            pl.BlockSpec((atoms_blk, hidden), lambda i: (i, 0)),
            pl.BlockSpec((atom_fdim, hidden), lambda i: (0, 0)),
            pl.BlockSpec((hidden, hidden), lambda i: (0, 0)),
            pl.BlockSpec((1, hidden), lambda i: (0, 0)),
        ],
        out_specs=pl.BlockSpec((mols_blk, hidden), lambda i: (i, 0)),
        out_shape=jax.ShapeDtypeStruct((n_mols, hidden), jnp.float32),
    )(fatoms, nei_atoms, W_o_aT, W_o_hT, bias)

    return sums / scope_lengths[:, None].astype(jnp.float32)


# speedup vs baseline: 1.9716x; 1.9716x over previous
import functools

import jax
import jax.numpy as jnp
from jax import lax
from jax.experimental import pallas as pl
from jax.experimental.pallas import tpu as pltpu
from jax.experimental.pallas import tpu_sc as plsc

LANES = 16


def _make_gather_sum(n_out, max_nb, hidden, inner=20, n_inner=2):
    mesh = plsc.VectorSubcoreMesh(core_axis_name="c", subcore_axis_name="s")
    nw = mesh.num_cores * mesh.num_subcores
    chunk = inner * n_inner
    per_worker = n_out // nw
    assert per_worker * nw == n_out and per_worker % 8 == 0
    n_chunks = per_worker // chunk
    assert n_chunks * chunk == per_worker and chunk % 8 == 0
    g_len = inner * max_nb
    assert g_len <= 128 and g_len % 8 == 0

    @functools.partial(
        pl.kernel,
        out_type=jax.ShapeDtypeStruct((n_out, hidden), jnp.float32),
        mesh=mesh,
        scratch_types=[
            pltpu.VMEM((chunk * max_nb,), jnp.int32),
            pltpu.VMEM((g_len, hidden), jnp.float32),
            pltpu.VMEM((chunk, hidden), jnp.float32),
            pltpu.SemaphoreType.DMA,
        ],
    )
    def gather_sum(idx_hbm, table_hbm, out_hbm, idx_v, rows_v, out_v, sem):
        wid = lax.axis_index("s") * mesh.num_cores + lax.axis_index("c")
        w_base = wid * per_worker

        def body(t, carry):
            base = w_base + t * chunk
            pltpu.sync_copy(idx_hbm.at[pl.ds(base * max_nb, chunk * max_nb)],
                            idx_v)
            for half in range(n_inner):
                pltpu.async_copy(
                    table_hbm.at[idx_v.at[pl.ds(half * g_len, g_len)]],
                    rows_v, sem).wait()
                for i in range(inner):
                    for l in range(hidden // LANES):
                        sl = pl.ds(l * LANES, LANES)
                        acc = rows_v[i * max_nb, sl]
                        for j in range(1, max_nb):
                            acc = acc + rows_v[i * max_nb + j, sl]
                        out_v[half * inner + i, sl] = acc
            pltpu.sync_copy(out_v, out_hbm.at[pl.ds(base, chunk)])
            return carry

        lax.fori_loop(0, n_chunks, body, 0)

    return gather_sum


def _init_body(fb_ref, wt_ref, bin_ref, msg_ref):
    b = jnp.dot(fb_ref[...], wt_ref[...], preferred_element_type=jnp.float32)
    bin_ref[...] = b
    msg_ref[...] = jnp.maximum(b, 0.0)


def _step_body(nei_ref, bin_ref, wt_ref, msg_ref):
    h = jnp.dot(nei_ref[...], wt_ref[...], preferred_element_type=jnp.float32)
    msg_ref[...] = jnp.maximum(bin_ref[...] + h, 0.0)


def _readout_body(fa_ref, nei_ref, wat_ref, wht_ref, b_ref, out_ref, *, apm, mols_blk):
    h = jnp.dot(fa_ref[...], wat_ref[...], preferred_element_type=jnp.float32)
    h += jnp.dot(nei_ref[...], wht_ref[...], preferred_element_type=jnp.float32)
    h = jnp.maximum(h + b_ref[...], 0.0)
    rows = lax.broadcasted_iota(jnp.int32, (mols_blk, mols_blk * apm), 0)
    cols = lax.broadcasted_iota(jnp.int32, (mols_blk, mols_blk * apm), 1) // apm
    seg = (rows == cols).astype(jnp.float32)
    out_ref[...] = jnp.dot(seg, h, preferred_element_type=jnp.float32).reshape(
        1, mols_blk, h.shape[1])


def kernel(fatoms, fbonds, agraph, bgraph, scope_starts, scope_lengths,
           W_i, W_h, W_o_w, W_o_b):
    n_atoms, atom_fdim = fatoms.shape
    n_bonds, bond_in = fbonds.shape
    max_nb = bgraph.shape[1]
    hidden = W_h.shape[0]
    n_mols = scope_starts.shape[0]
    apm = n_atoms // n_mols
    depth = 4

    bflat = bgraph.reshape(-1)
    n_atoms_pad = ((n_atoms + 32 * 40 - 1) // (32 * 40)) * (32 * 40)
    aflat = jnp.concatenate(
        [agraph.reshape(-1),
         jnp.zeros(((n_atoms_pad - n_atoms) * max_nb,), dtype=jnp.int32)])
    W_iT = W_i.T
    W_hT = W_h.T
    W_o_aT = W_o_w[:, :atom_fdim].T
    W_o_hT = W_o_w[:, atom_fdim:].T
    bias = W_o_b.reshape(1, hidden)

    gather_bonds = _make_gather_sum(n_bonds, max_nb, hidden)
    gather_atoms = _make_gather_sum(n_atoms_pad, max_nb, hidden)

    blk = 1000
    grid_b = n_bonds // blk
    binput, message = pl.pallas_call(
        _init_body,
        grid=(grid_b,),
        in_specs=[
            pl.BlockSpec((blk, bond_in), lambda i: (i, 0)),
            pl.BlockSpec((bond_in, hidden), lambda i: (0, 0)),
        ],
        out_specs=[pl.BlockSpec((blk, hidden), lambda i: (i, 0))] * 2,
        out_shape=[jax.ShapeDtypeStruct((n_bonds, hidden), jnp.float32)] * 2,
    )(fbonds, W_iT)

    step_call = pl.pallas_call(
        _step_body,
        grid=(grid_b,),
        in_specs=[
            pl.BlockSpec((blk, hidden), lambda i: (i, 0)),
            pl.BlockSpec((blk, hidden), lambda i: (i, 0)),
            pl.BlockSpec((hidden, hidden), lambda i: (0, 0)),
        ],
        out_specs=pl.BlockSpec((blk, hidden), lambda i: (i, 0)),
        out_shape=jax.ShapeDtypeStruct((n_bonds, hidden), jnp.float32),
    )

    for _ in range(depth - 1):
        nei = gather_bonds(bflat, message)
        message = step_call(nei, binput, W_hT)

    nei_atoms = gather_atoms(aflat, message)[:n_atoms]

    mols_blk = 20
    atoms_blk = mols_blk * apm
    grid_a = n_atoms // atoms_blk
    sums = pl.pallas_call(
        functools.partial(_readout_body, apm=apm, mols_blk=mols_blk),
        grid=(grid_a,),
        in_specs=[
            pl.BlockSpec((atoms_blk, atom_fdim), lambda i: (i, 0)),
            pl.BlockSpec((atoms_blk, hidden), lambda i: (i, 0)),
            pl.BlockSpec((atom_fdim, hidden), lambda i: (0, 0)),
            pl.BlockSpec((hidden, hidden), lambda i: (0, 0)),
            pl.BlockSpec((1, hidden), lambda i: (0, 0)),
        ],
        out_specs=pl.BlockSpec((1, mols_blk, hidden), lambda i: (i, 0, 0)),
        out_shape=jax.ShapeDtypeStruct((grid_a, mols_blk, hidden), jnp.float32),
    )(fatoms, nei_atoms, W_o_aT, W_o_hT, bias)

    return sums.reshape(n_mols, hidden) / scope_lengths[:, None].astype(jnp.float32)


# speedup vs baseline: 3.7471x; 1.9006x over previous
import functools

import jax
import jax.numpy as jnp
from jax import lax
from jax.experimental import pallas as pl
from jax.experimental.pallas import tpu as pltpu
from jax.experimental.pallas import tpu_sc as plsc

LANES = 16


def _make_gather_sum(n_out, max_nb, hidden, inner=20, n_inner=2):
    mesh = plsc.VectorSubcoreMesh(core_axis_name="c", subcore_axis_name="s")
    nw = mesh.num_cores * mesh.num_subcores
    chunk = inner * n_inner
    per_worker = n_out // nw
    assert per_worker * nw == n_out and per_worker % 8 == 0
    n_chunks = per_worker // chunk
    assert n_chunks * chunk == per_worker and chunk % 8 == 0
    g_len = inner * max_nb
    assert g_len <= 128 and g_len % 8 == 0

    T = n_chunks
    cb = chunk * max_nb
    assert T >= 4

    @functools.partial(
        pl.kernel,
        out_type=jax.ShapeDtypeStruct((n_out, hidden), jnp.float32),
        mesh=mesh,
        scratch_types=[
            pltpu.VMEM((cb,), jnp.int32),
            pltpu.VMEM((cb,), jnp.int32),
            pltpu.VMEM((cb, hidden), jnp.float32),
            pltpu.VMEM((cb, hidden), jnp.float32),
            pltpu.VMEM((chunk, hidden), jnp.float32),
            pltpu.VMEM((chunk, hidden), jnp.float32),
            pltpu.SemaphoreType.DMA,
            pltpu.SemaphoreType.DMA,
            pltpu.SemaphoreType.DMA,
            pltpu.SemaphoreType.DMA,
            pltpu.SemaphoreType.DMA,
            pltpu.SemaphoreType.DMA,
        ],
    )
    def gather_sum(idx_hbm, table_hbm, out_hbm, idx_v0, idx_v1, rows_v0,
                   rows_v1, out_v0, out_v1, si0, si1, sg0, sg1, so0, so1):
        wid = lax.axis_index("s") * mesh.num_cores + lax.axis_index("c")
        w_base = wid * per_worker
        idx_b, rows_b, out_b = [idx_v0, idx_v1], [rows_v0, rows_v1], [out_v0, out_v1]
        si_b, sg_b, so_b = [si0, si1], [sg0, sg1], [so0, so1]

        def idx_copy(t, b):
            return pltpu.make_async_copy(
                idx_hbm.at[pl.ds((w_base + t * chunk) * max_nb, cb)],
                idx_b[b], si_b[b])

        def g_copy(t, b, half):
            sl = pl.ds(half * g_len, g_len)
            return pltpu.make_async_copy(
                table_hbm.at[idx_b[b].at[sl]], rows_b[b].at[sl], sg_b[b])

        def o_copy(t, b):
            return pltpu.make_async_copy(
                out_b[b],
                out_hbm.at[pl.ds(w_base + t * chunk, chunk)], so_b[b])

        def do_chunk(t, b, last):
            if not last:
                @pl.when(t + 1 < T)
                def _():
                    idx_copy(t + 1, 1 - b).wait()
                    for h in range(n_inner):
                        g_copy(t + 1, 1 - b, h).start()
            for h in range(n_inner):
                g_copy(t, b, h).wait()
            if not last:
                @pl.when(t + 2 < T)
                def _():
                    idx_copy(t + 2, b).start()

            @pl.when(t >= 2)
            def _():
                o_copy(t - 2, b).wait()

            rows, out = rows_b[b], out_b[b]
            u = 8

            def sum_body(g, carry):
                i0 = g * u
                for di in range(u):
                    for l in range(hidden // LANES):
                        sl = pl.ds(l * LANES, LANES)
                        acc = rows[(i0 + di) * max_nb, sl]
                        for j in range(1, max_nb):
                            acc = acc + rows[(i0 + di) * max_nb + j, sl]
                        out[i0 + di, sl] = acc
                return carry

            lax.fori_loop(0, chunk // u, sum_body, 0)
            o_copy(t, b).start()

        pltpu.sync_copy(idx_hbm.at[pl.ds(w_base * max_nb, cb)], idx_v0)
        for h in range(n_inner):
            g_copy(0, 0, h).start()
        idx_copy(1, 1).start()

        T_main = T - (T % 2)

        def body(t2, carry):
            do_chunk(2 * t2, 0, last=False)
            do_chunk(2 * t2 + 1, 1, last=False)
            return carry

        lax.fori_loop(0, T_main // 2, body, 0)
        if T % 2:
            do_chunk(T - 1, (T - 1) % 2, last=True)
        o_copy(T - 2, (T - 2) % 2).wait()
        o_copy(T - 1, (T - 1) % 2).wait()

    return gather_sum


def _init_body(fb_ref, wt_ref, bin_ref, msg_ref):
    b = jnp.dot(fb_ref[...], wt_ref[...], preferred_element_type=jnp.float32)
    bin_ref[...] = b
    msg_ref[...] = jnp.maximum(b, 0.0)


def _step_body(nei_ref, bin_ref, wt_ref, msg_ref):
    h = jnp.dot(nei_ref[...], wt_ref[...], preferred_element_type=jnp.float32)
    msg_ref[...] = jnp.maximum(bin_ref[...] + h, 0.0)


def _readout_body(fa_ref, nei_ref, wat_ref, wht_ref, b_ref, out_ref, *, apm, mols_blk):
    h = jnp.dot(fa_ref[...], wat_ref[...], preferred_element_type=jnp.float32)
    h += jnp.dot(nei_ref[...], wht_ref[...], preferred_element_type=jnp.float32)
    h = jnp.maximum(h + b_ref[...], 0.0)
    rows = lax.broadcasted_iota(jnp.int32, (mols_blk, mols_blk * apm), 0)
    cols = lax.broadcasted_iota(jnp.int32, (mols_blk, mols_blk * apm), 1) // apm
    seg = (rows == cols).astype(jnp.float32)
    out_ref[...] = jnp.dot(seg, h, preferred_element_type=jnp.float32).reshape(
        1, mols_blk, h.shape[1])


def kernel(fatoms, fbonds, agraph, bgraph, scope_starts, scope_lengths,
           W_i, W_h, W_o_w, W_o_b):
    n_atoms, atom_fdim = fatoms.shape
    n_bonds, bond_in = fbonds.shape
    max_nb = bgraph.shape[1]
    hidden = W_h.shape[0]
    n_mols = scope_starts.shape[0]
    apm = n_atoms // n_mols
    depth = 4

    bflat = bgraph.reshape(-1)
    n_atoms_pad = ((n_atoms + 32 * 40 - 1) // (32 * 40)) * (32 * 40)
    aflat = jnp.concatenate(
        [agraph.reshape(-1),
         jnp.zeros(((n_atoms_pad - n_atoms) * max_nb,), dtype=jnp.int32)])
    W_iT = W_i.T
    W_hT = W_h.T
    W_o_aT = W_o_w[:, :atom_fdim].T
    W_o_hT = W_o_w[:, atom_fdim:].T
    bias = W_o_b.reshape(1, hidden)

    gather_bonds = _make_gather_sum(n_bonds, max_nb, hidden)
    gather_atoms = _make_gather_sum(n_atoms_pad, max_nb, hidden)

    blk = 1000
    grid_b = n_bonds // blk
    binput, message = pl.pallas_call(
        _init_body,
        grid=(grid_b,),
        in_specs=[
            pl.BlockSpec((blk, bond_in), lambda i: (i, 0)),
            pl.BlockSpec((bond_in, hidden), lambda i: (0, 0)),
        ],
        out_specs=[pl.BlockSpec((blk, hidden), lambda i: (i, 0))] * 2,
        out_shape=[jax.ShapeDtypeStruct((n_bonds, hidden), jnp.float32)] * 2,
    )(fbonds, W_iT)

    step_call = pl.pallas_call(
        _step_body,
        grid=(grid_b,),
        in_specs=[
            pl.BlockSpec((blk, hidden), lambda i: (i, 0)),
            pl.BlockSpec((blk, hidden), lambda i: (i, 0)),
            pl.BlockSpec((hidden, hidden), lambda i: (0, 0)),
        ],
        out_specs=pl.BlockSpec((blk, hidden), lambda i: (i, 0)),
        out_shape=jax.ShapeDtypeStruct((n_bonds, hidden), jnp.float32),
    )

    for _ in range(depth - 1):
        nei = gather_bonds(bflat, message)
        message = step_call(nei, binput, W_hT)

    nei_atoms = gather_atoms(aflat, message)[:n_atoms]

    mols_blk = 20
    atoms_blk = mols_blk * apm
    grid_a = n_atoms // atoms_blk
    sums = pl.pallas_call(
        functools.partial(_readout_body, apm=apm, mols_blk=mols_blk),
        grid=(grid_a,),
        in_specs=[
            pl.BlockSpec((atoms_blk, atom_fdim), lambda i: (i, 0)),
            pl.BlockSpec((atoms_blk, hidden), lambda i: (i, 0)),
            pl.BlockSpec((atom_fdim, hidden), lambda i: (0, 0)),
            pl.BlockSpec((hidden, hidden), lambda i: (0, 0)),
            pl.BlockSpec((1, hidden), lambda i: (0, 0)),
        ],
        out_specs=pl.BlockSpec((1, mols_blk, hidden), lambda i: (i, 0, 0)),
        out_shape=jax.ShapeDtypeStruct((grid_a, mols_blk, hidden), jnp.float32),
    )(fatoms, nei_atoms, W_o_aT, W_o_hT, bias)

    return sums.reshape(n_mols, hidden) / scope_lengths[:, None].astype(jnp.float32)


# speedup vs baseline: 4.0793x; 1.0886x over previous
import functools

import jax
import jax.numpy as jnp
from jax import lax
from jax.experimental import pallas as pl
from jax.experimental.pallas import tpu as pltpu
from jax.experimental.pallas import tpu_sc as plsc

LANES = 16


def _make_gather_sum(n_out, max_nb, hidden, inner=20, n_inner=2):
    mesh = plsc.VectorSubcoreMesh(core_axis_name="c", subcore_axis_name="s")
    nw = mesh.num_cores * mesh.num_subcores
    chunk = inner * n_inner
    per_worker = n_out // nw
    n_chunks = per_worker // chunk
    g_len = inner * max_nb
    assert per_worker * nw == n_out and per_worker % 8 == 0
    assert n_chunks * chunk == per_worker and chunk % 8 == 0
    assert g_len <= 128 and g_len % 8 == 0
    T = n_chunks
    cb = chunk * max_nb
    assert T >= 4

    @functools.partial(
        pl.kernel,
        out_type=jax.ShapeDtypeStruct((n_out, hidden), jnp.float32),
        mesh=mesh,
        scratch_types=[
            pltpu.VMEM((cb,), jnp.int32),
            pltpu.VMEM((cb,), jnp.int32),
            pltpu.VMEM((cb, hidden), jnp.float32),
            pltpu.VMEM((cb, hidden), jnp.float32),
            pltpu.VMEM((chunk, hidden), jnp.float32),
            pltpu.VMEM((chunk, hidden), jnp.float32),
            pltpu.SemaphoreType.DMA,
            pltpu.SemaphoreType.DMA,
            pltpu.SemaphoreType.DMA,
            pltpu.SemaphoreType.DMA,
            pltpu.SemaphoreType.DMA,
            pltpu.SemaphoreType.DMA,
        ],
    )
    def gather_sum(idx_hbm, table_hbm, out_hbm, idx_v0, idx_v1, rows_v0,
                   rows_v1, out_v0, out_v1, si0, si1, sg0, sg1, so0, so1):
        wid = lax.axis_index("s") * mesh.num_cores + lax.axis_index("c")
        w_base = wid * per_worker
        idx_b, rows_b, out_b = [idx_v0, idx_v1], [rows_v0, rows_v1], [out_v0, out_v1]
        si_b, sg_b, so_b = [si0, si1], [sg0, sg1], [so0, so1]

        def idx_copy(t, b):
            return pltpu.make_async_copy(
                idx_hbm.at[pl.ds((w_base + t * chunk) * max_nb, cb)],
                idx_b[b], si_b[b])

        def g_copy(t, b, half):
            sl = pl.ds(half * g_len, g_len)
            return pltpu.make_async_copy(
                table_hbm.at[idx_b[b].at[sl]], rows_b[b].at[sl], sg_b[b])

        def o_copy(t, b):
            return pltpu.make_async_copy(
                out_b[b],
                out_hbm.at[pl.ds(w_base + t * chunk, chunk)], so_b[b])

        def do_chunk(t, b, last):
            if not last:
                @pl.when(t + 1 < T)
                def _():
                    idx_copy(t + 1, 1 - b).wait()
                    for h in range(n_inner):
                        g_copy(t + 1, 1 - b, h).start()
            for h in range(n_inner):
                g_copy(t, b, h).wait()
            if not last:
                @pl.when(t + 2 < T)
                def _():
                    idx_copy(t + 2, b).start()

            @pl.when(t >= 2)
            def _():
                o_copy(t - 2, b).wait()

            rows, out = rows_b[b], out_b[b]
            u = 8

            def sum_body(g, carry):
                i0 = g * u
                for di in range(u):
                    for l in range(hidden // LANES):
                        sl = pl.ds(l * LANES, LANES)
                        acc = rows[(i0 + di) * max_nb, sl]
                        for j in range(1, max_nb):
                            acc = acc + rows[(i0 + di) * max_nb + j, sl]
                        out[i0 + di, sl] = acc
                return carry

            lax.fori_loop(0, chunk // u, sum_body, 0)
            o_copy(t, b).start()

        pltpu.sync_copy(idx_hbm.at[pl.ds(w_base * max_nb, cb)], idx_v0)
        for h in range(n_inner):
            g_copy(0, 0, h).start()
        idx_copy(1, 1).start()

        T_main = T - (T % 2)

        def body(t2, carry):
            do_chunk(2 * t2, 0, last=False)
            do_chunk(2 * t2 + 1, 1, last=False)
            return carry

        lax.fori_loop(0, T_main // 2, body, 0)
        if T % 2:
            do_chunk(T - 1, (T - 1) % 2, last=True)
        o_copy(T - 2, (T - 2) % 2).wait()
        o_copy(T - 1, (T - 1) % 2).wait()

    return gather_sum


def _init_body(fb_ref, wt_ref, bin_ref, msg_ref):
    b = jnp.dot(fb_ref[...], wt_ref[...], preferred_element_type=jnp.float32)
    bin_ref[...] = b
    msg_ref[...] = jnp.maximum(b, 0.0)


def _step_body(nei_ref, bin_ref, wt_ref, msg_ref):
    h = jnp.dot(nei_ref[...].astype(jnp.bfloat16), wt_ref[...],
                preferred_element_type=jnp.float32)
    msg_ref[...] = jnp.maximum(bin_ref[...] + h, 0.0)


def _readout_body(fa_ref, nei_ref, wat_ref, wht_ref, b_ref, out_ref, *, apm, mols_blk):
    h = jnp.dot(fa_ref[...], wat_ref[...], preferred_element_type=jnp.float32)
    h += jnp.dot(nei_ref[...].astype(jnp.bfloat16), wht_ref[...],
                 preferred_element_type=jnp.float32)
    h = jnp.maximum(h + b_ref[...], 0.0)
    rows = lax.broadcasted_iota(jnp.int32, (mols_blk, mols_blk * apm), 0)
    cols = lax.broadcasted_iota(jnp.int32, (mols_blk, mols_blk * apm), 1) // apm
    seg = (rows == cols).astype(jnp.float32)
    out_ref[...] = jnp.dot(seg, h, preferred_element_type=jnp.float32).reshape(
        1, mols_blk, h.shape[1])


def kernel(fatoms, fbonds, agraph, bgraph, scope_starts, scope_lengths,
           W_i, W_h, W_o_w, W_o_b):
    n_atoms, atom_fdim = fatoms.shape
    n_bonds, bond_in = fbonds.shape
    max_nb = bgraph.shape[1]
    hidden = W_h.shape[0]
    n_mols = scope_starts.shape[0]
    apm = n_atoms // n_mols
    depth = 4

    bflat = bgraph.reshape(-1)
    gran = 32 * 40
    n_atoms_pad = ((n_atoms + gran - 1) // gran) * gran
    aflat = jnp.concatenate(
        [agraph.reshape(-1),
         jnp.zeros(((n_atoms_pad - n_atoms) * max_nb,), dtype=jnp.int32)])
    W_iT = W_i.T
    W_hT = W_h.T.astype(jnp.bfloat16)
    W_o_aT = W_o_w[:, :atom_fdim].T
    W_o_hT = W_o_w[:, atom_fdim:].T.astype(jnp.bfloat16)
    bias = W_o_b.reshape(1, hidden)

    gather_bonds = _make_gather_sum(n_bonds, max_nb, hidden)
    gather_atoms = _make_gather_sum(n_atoms_pad, max_nb, hidden)

    blk = 2000
    grid_b = n_bonds // blk
    binput, message = pl.pallas_call(
        _init_body,
        grid=(grid_b,),
        in_specs=[
            pl.BlockSpec((blk, bond_in), lambda i: (i, 0)),
            pl.BlockSpec((bond_in, hidden), lambda i: (0, 0)),
        ],
        out_specs=[pl.BlockSpec((blk, hidden), lambda i: (i, 0))] * 2,
        out_shape=[jax.ShapeDtypeStruct((n_bonds, hidden), jnp.float32)] * 2,
    )(fbonds, W_iT)

    step_call = pl.pallas_call(
        _step_body,
        grid=(grid_b,),
        in_specs=[
            pl.BlockSpec((blk, hidden), lambda i: (i, 0)),
            pl.BlockSpec((blk, hidden), lambda i: (i, 0)),
            pl.BlockSpec((hidden, hidden), lambda i: (0, 0)),
        ],
        out_specs=pl.BlockSpec((blk, hidden), lambda i: (i, 0)),
        out_shape=jax.ShapeDtypeStruct((n_bonds, hidden), jnp.float32),
    )

    for _ in range(depth - 1):
        nei = gather_bonds(bflat, message)
        message = step_call(nei, binput, W_hT)

    nei_atoms = gather_atoms(aflat, message)[:n_atoms]

    mols_blk = 20
    atoms_blk = mols_blk * apm
    grid_a = n_atoms // atoms_blk
    sums = pl.pallas_call(
        functools.partial(_readout_body, apm=apm, mols_blk=mols_blk),
        grid=(grid_a,),
        in_specs=[
            pl.BlockSpec((atoms_blk, atom_fdim), lambda i: (i, 0)),
            pl.BlockSpec((atoms_blk, hidden), lambda i: (i, 0)),
            pl.BlockSpec((atom_fdim, hidden), lambda i: (0, 0)),
            pl.BlockSpec((hidden, hidden), lambda i: (0, 0)),
            pl.BlockSpec((1, hidden), lambda i: (0, 0)),
        ],
        out_specs=pl.BlockSpec((1, mols_blk, hidden), lambda i: (i, 0, 0)),
        out_shape=jax.ShapeDtypeStruct((grid_a, mols_blk, hidden), jnp.float32),
    )(fatoms, nei_atoms, W_o_aT, W_o_hT, bias)

    return sums.reshape(n_mols, hidden) / scope_lengths[:, None].astype(jnp.float32)
